# baseline (speedup 1.0000x reference)
import dataclasses
import functools

import numpy as np
import jax
import jax.numpy as jnp
from jax import lax
from jax.experimental import pallas as pl
from jax.experimental.pallas import tpu as pltpu
from jax.experimental.pallas import tpu_sc as plsc

_NUM_LEVELS = 16
_N = 262144
_HASH_ROWS = 4096
_MASK = np.int32(4095)
_P1 = np.int32(np.uint32(2654435761).view(np.int32))
_P2 = np.int32(805459861)
_RES = [int(np.ceil(16 * 2.0 ** i)) for i in range(_NUM_LEVELS)]

_NW = 32
_PER_W = _N // _NW
_CH = 256
_NPAIR = _PER_W // (2 * _CH)
_S = 272
_HI16 = np.int32(np.uint32(0xFFFF0000).view(np.int32))


def _lerp(a, b, w, one_minus_w):
    return a * one_minus_w + b * w


def _f32_hi(word):
    return lax.bitcast_convert_type(word & _HI16, jnp.float32)


def _f32_lo(word):
    return lax.bitcast_convert_type(lax.shift_left(word, 16), jnp.float32)


def _encode_body(px_hbm, py_hbm, pz_hbm, tp_hbm, out_hbm,
                 px_v, py_v, pz_v, t_v, o_v, o_r0, o_r1, sem_p, sem0, sem1):
    cid = lax.axis_index("c")
    sid = lax.axis_index("s")
    wid = sid * 2 + cid
    base_w = wid * _PER_W

    pltpu.async_copy(px_hbm.at[pl.ds(base_w, _PER_W)], px_v, sem_p)
    pltpu.async_copy(py_hbm.at[pl.ds(base_w, _PER_W)], py_v, sem_p)
    pltpu.async_copy(pz_hbm.at[pl.ds(base_w, _PER_W)], pz_v, sem_p)
    pltpu.async_copy(tp_hbm, t_v, sem_p).wait()
    pltpu.make_async_copy(px_hbm.at[pl.ds(base_w, _PER_W)], px_v, sem_p).wait()
    pltpu.make_async_copy(py_hbm.at[pl.ds(base_w, _PER_W)], py_v, sem_p).wait()
    pltpu.make_async_copy(pz_hbm.at[pl.ds(base_w, _PER_W)], pz_v, sem_p).wait()

    iota = lax.iota(jnp.int32, 16)
    iota_lo = iota * _S
    iota_hi = iota_lo + 16 * _S

    def compute_chunk(ch, o_r):
        coff = ch * _CH

        @pl.loop(0, _CH // 16)
        def _(pb):
            off = pb * 16
            x = px_v[pl.ds(coff + off, 16)]
            y = py_v[pl.ds(coff + off, 16)]
            z = pz_v[pl.ds(coff + off, 16)]
            for l in range(_NUM_LEVELS):
                rf = np.float32(_RES[l] - 1)
                rm1 = np.int32(_RES[l] - 1)
                sx = (x + 1.0) * 0.5 * rf
                sy = (y + 1.0) * 0.5 * rf
                sz = (z + 1.0) * 0.5 * rf
                ix = sx.astype(jnp.int32)
                iy = sy.astype(jnp.int32)
                iz = sz.astype(jnp.int32)
                wx = sx - ix.astype(jnp.float32)
                wy = sy - iy.astype(jnp.float32)
                wz = sz - iz.astype(jnp.float32)
                x1 = jnp.minimum(ix + 1, rm1)
                y1 = jnp.minimum(iy + 1, rm1)
                z1 = jnp.minimum(iz + 1, rm1)
                hy0 = iy * _P1
                hy1 = y1 * _P1
                hz0 = iz * _P2
                hz1 = z1 * _P2
                e00 = ix ^ hy0
                e01 = ix ^ hy1
                e10 = x1 ^ hy0
                e11 = x1 ^ hy1
                h = [
                    (e00 ^ hz0) & _MASK,
                    (e00 ^ hz1) & _MASK,
                    (e01 ^ hz0) & _MASK,
                    (e01 ^ hz1) & _MASK,
                    (e10 ^ hz0) & _MASK,
                    (e10 ^ hz1) & _MASK,
                    (e11 ^ hz0) & _MASK,
                    (e11 ^ hz1) & _MASK,
                ]
                tl = t_v.at[pl.ds(l * _HASH_ROWS, _HASH_ROWS)]
                g = [plsc.bitcast(plsc.load_gather(tl, [hj]), jnp.bfloat16)
                     for hj in h]
                fmt = plsc.PackFormat.INTERLEAVED
                wxp = plsc.pack(wx, wx, format=fmt)
                wyp = plsc.pack(wy, wy, format=fmt)
                wzp = plsc.pack(wz, wz, format=fmt)
                owx = 1.0 - wxp
                owy = 1.0 - wyp
                owz = 1.0 - wzp
                c00 = _lerp(g[0], g[1], wxp, owx)
                c01 = _lerp(g[2], g[3], wxp, owx)
                c10 = _lerp(g[4], g[5], wxp, owx)
                c11 = _lerp(g[6], g[7], wxp, owx)
                c0 = _lerp(c00, c01, wyp, owy)
                c1 = _lerp(c10, c11, wyp, owy)
                val = _lerp(c0, c1, wzp, owz)
                w = plsc.bitcast(val, jnp.int32)
                o_v[pl.ds((2 * l) * _S + off, 16)] = _f32_lo(w)
                o_v[pl.ds((2 * l + 1) * _S + off, 16)] = _f32_hi(w)

        @pl.loop(0, _CH // 4)
        def _(t):
            for k in range(4):
                p = t * 4 + k
                o_r[p, pl.ds(0, 16)] = plsc.load_gather(o_v, [iota_lo + p])
                o_r[p, pl.ds(16, 16)] = plsc.load_gather(o_v, [iota_hi + p])

    def out_slice(ch):
        return out_hbm.at[pl.ds(base_w + ch * _CH, _CH), :]

    @pl.loop(0, _NPAIR)
    def _(pair):
        ch0 = pair * 2

        @pl.when(pair > 0)
        def _():
            pltpu.make_async_copy(o_r0, out_slice(ch0), sem0).wait()

        compute_chunk(ch0, o_r0)
        pltpu.async_copy(o_r0, out_slice(ch0), sem0)

        @pl.when(pair > 0)
        def _():
            pltpu.make_async_copy(o_r1, out_slice(ch0 + 1), sem1).wait()

        compute_chunk(ch0 + 1, o_r1)
        pltpu.async_copy(o_r1, out_slice(ch0 + 1), sem1)

    last = 2 * (_NPAIR - 1)
    pltpu.make_async_copy(o_r0, out_slice(last), sem0).wait()
    pltpu.make_async_copy(o_r1, out_slice(last + 1), sem1).wait()


@jax.jit
def _sc_encode(px, py, pz, tp):
    mesh = plsc.VectorSubcoreMesh(core_axis_name="c", subcore_axis_name="s")
    cp = pltpu.CompilerParams()
    for fld, val in (("needs_layout_passes", False),
                     ("use_tc_tiling_on_sc", False)):
        if fld in pltpu.CompilerParams.__dataclass_fields__:
            cp = dataclasses.replace(cp, **{fld: val})
    f = functools.partial(
        pl.kernel,
        compiler_params=cp,
        out_type=jax.ShapeDtypeStruct((_N, 2 * _NUM_LEVELS), jnp.float32),
        mesh=mesh,
        scratch_types=[
            pltpu.VMEM((_PER_W,), jnp.float32),
            pltpu.VMEM((_PER_W,), jnp.float32),
            pltpu.VMEM((_PER_W,), jnp.float32),
            pltpu.VMEM((_NUM_LEVELS * _HASH_ROWS,), jnp.int32),
            pltpu.VMEM((2 * _NUM_LEVELS * _S,), jnp.float32),
            pltpu.VMEM((_CH, 2 * _NUM_LEVELS), jnp.float32),
            pltpu.VMEM((_CH, 2 * _NUM_LEVELS), jnp.float32),
            pltpu.SemaphoreType.DMA,
            pltpu.SemaphoreType.DMA,
            pltpu.SemaphoreType.DMA,
        ],
    )(_encode_body)
    return f(px, py, pz, tp)


def kernel(positions, table_0, table_1, table_2, table_3, table_4, table_5,
           table_6, table_7, table_8, table_9, table_10, table_11, table_12,
           table_13, table_14, table_15):
    tables = [table_0, table_1, table_2, table_3, table_4, table_5, table_6,
              table_7, table_8, table_9, table_10, table_11, table_12,
              table_13, table_14, table_15]
    px = positions[:, 0]
    py = positions[:, 1]
    pz = positions[:, 2]
    packed = []
    for t in tables:
        bits = lax.bitcast_convert_type(
            t[:_HASH_ROWS].astype(jnp.bfloat16), jnp.uint16)
        word = bits[:, 0].astype(jnp.uint32) | (
            bits[:, 1].astype(jnp.uint32) << 16)
        packed.append(lax.bitcast_convert_type(word, jnp.int32))
    tp = jnp.concatenate(packed)
    return _sc_encode(px, py, pz, tp)


# speedup vs baseline: 1.8485x; 1.8485x over previous
import dataclasses
import functools

import numpy as np
import jax
import jax.numpy as jnp
from jax import lax
from jax.experimental import pallas as pl
from jax.experimental.pallas import tpu as pltpu
from jax.experimental.pallas import tpu_sc as plsc

_NUM_LEVELS = 16
_N = 262144
_HASH_ROWS = 4096
_MASK = np.int32(4095)
_P1 = np.int32(np.uint32(2654435761).view(np.int32))
_P2 = np.int32(805459861)
_RES = [int(np.ceil(16 * 2.0 ** i)) for i in range(_NUM_LEVELS)]

_NW = 32
_PER_W = _N // _NW
_CH = 512
_NPAIR = _PER_W // (2 * _CH)
_HI16 = np.int32(np.uint32(0xFFFF0000).view(np.int32))


def _lerp(a, b, w, one_minus_w):
    return a * one_minus_w + b * w


def _f32_hi(word):
    return lax.bitcast_convert_type(word & _HI16, jnp.float32)


def _f32_lo(word):
    return lax.bitcast_convert_type(lax.shift_left(word, 16), jnp.float32)


def _encode_body(px_hbm, py_hbm, pz_hbm, tp_hbm, out_hbm,
                 px_v, py_v, pz_v, t_v, o_v0, o_v1, sem_p, sem0, sem1):
    cid = lax.axis_index("c")
    sid = lax.axis_index("s")
    wid = sid * 2 + cid
    base_w = wid * _PER_W

    pltpu.async_copy(px_hbm.at[pl.ds(base_w, _PER_W)], px_v, sem_p)
    pltpu.async_copy(py_hbm.at[pl.ds(base_w, _PER_W)], py_v, sem_p)
    pltpu.async_copy(pz_hbm.at[pl.ds(base_w, _PER_W)], pz_v, sem_p)
    pltpu.async_copy(tp_hbm, t_v, sem_p).wait()
    pltpu.make_async_copy(px_hbm.at[pl.ds(base_w, _PER_W)], px_v, sem_p).wait()
    pltpu.make_async_copy(py_hbm.at[pl.ds(base_w, _PER_W)], py_v, sem_p).wait()
    pltpu.make_async_copy(pz_hbm.at[pl.ds(base_w, _PER_W)], pz_v, sem_p).wait()

    def compute_chunk(ch, o_v):
        coff = ch * _CH

        @pl.loop(0, _CH // 16)
        def _(pb):
            off = pb * 16
            x = px_v[pl.ds(coff + off, 16)]
            y = py_v[pl.ds(coff + off, 16)]
            z = pz_v[pl.ds(coff + off, 16)]
            for l in range(_NUM_LEVELS):
                rf = np.float32(_RES[l] - 1)
                rm1 = np.int32(_RES[l] - 1)
                sx = (x + 1.0) * 0.5 * rf
                sy = (y + 1.0) * 0.5 * rf
                sz = (z + 1.0) * 0.5 * rf
                ix = sx.astype(jnp.int32)
                iy = sy.astype(jnp.int32)
                iz = sz.astype(jnp.int32)
                wx = sx - ix.astype(jnp.float32)
                wy = sy - iy.astype(jnp.float32)
                wz = sz - iz.astype(jnp.float32)
                x1 = jnp.minimum(ix + 1, rm1)
                y1 = jnp.minimum(iy + 1, rm1)
                z1 = jnp.minimum(iz + 1, rm1)
                hy0 = iy * _P1
                hy1 = y1 * _P1
                hz0 = iz * _P2
                hz1 = z1 * _P2
                e00 = ix ^ hy0
                e01 = ix ^ hy1
                e10 = x1 ^ hy0
                e11 = x1 ^ hy1
                h = [
                    (e00 ^ hz0) & _MASK,
                    (e00 ^ hz1) & _MASK,
                    (e01 ^ hz0) & _MASK,
                    (e01 ^ hz1) & _MASK,
                    (e10 ^ hz0) & _MASK,
                    (e10 ^ hz1) & _MASK,
                    (e11 ^ hz0) & _MASK,
                    (e11 ^ hz1) & _MASK,
                ]
                tl = t_v.at[pl.ds(l * _HASH_ROWS, _HASH_ROWS)]
                g = [plsc.bitcast(plsc.load_gather(tl, [hj]), jnp.bfloat16)
                     for hj in h]
                fmt = plsc.PackFormat.INTERLEAVED
                wxp = plsc.pack(wx, wx, format=fmt)
                wyp = plsc.pack(wy, wy, format=fmt)
                wzp = plsc.pack(wz, wz, format=fmt)
                owx = 1.0 - wxp
                owy = 1.0 - wyp
                owz = 1.0 - wzp
                c00 = _lerp(g[0], g[1], wxp, owx)
                c01 = _lerp(g[2], g[3], wxp, owx)
                c10 = _lerp(g[4], g[5], wxp, owx)
                c11 = _lerp(g[6], g[7], wxp, owx)
                c0 = _lerp(c00, c01, wyp, owy)
                c1 = _lerp(c10, c11, wyp, owy)
                val = _lerp(c0, c1, wzp, owz)
                w = plsc.bitcast(val, jnp.int32)
                o_v[2 * l, pl.ds(off, 16)] = _f32_lo(w)
                o_v[2 * l + 1, pl.ds(off, 16)] = _f32_hi(w)

    def out_slice(ch):
        return out_hbm.at[:, pl.ds(base_w + ch * _CH, _CH)]

    @pl.loop(0, _NPAIR)
    def _(pair):
        ch0 = pair * 2

        @pl.when(pair > 0)
        def _():
            pltpu.make_async_copy(o_v0, out_slice(ch0), sem0).wait()

        compute_chunk(ch0, o_v0)
        pltpu.async_copy(o_v0, out_slice(ch0), sem0)

        @pl.when(pair > 0)
        def _():
            pltpu.make_async_copy(o_v1, out_slice(ch0 + 1), sem1).wait()

        compute_chunk(ch0 + 1, o_v1)
        pltpu.async_copy(o_v1, out_slice(ch0 + 1), sem1)

    last = 2 * (_NPAIR - 1)
    pltpu.make_async_copy(o_v0, out_slice(last), sem0).wait()
    pltpu.make_async_copy(o_v1, out_slice(last + 1), sem1).wait()


@jax.jit
def _sc_encode(px, py, pz, tp):
    mesh = plsc.VectorSubcoreMesh(core_axis_name="c", subcore_axis_name="s")
    cp = pltpu.CompilerParams()
    for fld, val in (("needs_layout_passes", False),
                     ("use_tc_tiling_on_sc", False)):
        if fld in pltpu.CompilerParams.__dataclass_fields__:
            cp = dataclasses.replace(cp, **{fld: val})
    f = functools.partial(
        pl.kernel,
        compiler_params=cp,
        out_type=jax.ShapeDtypeStruct((2 * _NUM_LEVELS, _N), jnp.float32),
        mesh=mesh,
        scratch_types=[
            pltpu.VMEM((_PER_W,), jnp.float32),
            pltpu.VMEM((_PER_W,), jnp.float32),
            pltpu.VMEM((_PER_W,), jnp.float32),
            pltpu.VMEM((_NUM_LEVELS * _HASH_ROWS,), jnp.int32),
            pltpu.VMEM((2 * _NUM_LEVELS, _CH), jnp.float32),
            pltpu.VMEM((2 * _NUM_LEVELS, _CH), jnp.float32),
            pltpu.SemaphoreType.DMA,
            pltpu.SemaphoreType.DMA,
            pltpu.SemaphoreType.DMA,
        ],
    )(_encode_body)
    out_t = f(px, py, pz, tp)
    return out_t


def kernel(positions, table_0, table_1, table_2, table_3, table_4, table_5,
           table_6, table_7, table_8, table_9, table_10, table_11, table_12,
           table_13, table_14, table_15):
    tables = [table_0, table_1, table_2, table_3, table_4, table_5, table_6,
              table_7, table_8, table_9, table_10, table_11, table_12,
              table_13, table_14, table_15]
    px = positions[:, 0]
    py = positions[:, 1]
    pz = positions[:, 2]
    packed = []
    for t in tables:
        bits = lax.bitcast_convert_type(
            t[:_HASH_ROWS].astype(jnp.bfloat16), jnp.uint16)
        word = bits[:, 0].astype(jnp.uint32) | (
            bits[:, 1].astype(jnp.uint32) << 16)
        packed.append(lax.bitcast_convert_type(word, jnp.int32))
    tp = jnp.concatenate(packed)
    return _sc_encode(px, py, pz, tp)


# speedup vs baseline: 1.9630x; 1.0620x over previous
import dataclasses
import functools

import numpy as np
import jax
import jax.numpy as jnp
from jax import lax
from jax.experimental import pallas as pl
from jax.experimental.pallas import tpu as pltpu
from jax.experimental.pallas import tpu_sc as plsc

_NUM_LEVELS = 16
_N = 262144
_HASH_ROWS = 4096
_MASK = np.int32(4095)
_P1 = np.int32(np.uint32(2654435761).view(np.int32))
_P2 = np.int32(805459861)
_RES = [int(np.ceil(16 * 2.0 ** i)) for i in range(_NUM_LEVELS)]

_NW = 32
_PER_W = _N // _NW
_CH = 512
_NPAIR = _PER_W // (2 * _CH)
_HI16 = np.int32(np.uint32(0xFFFF0000).view(np.int32))


def _lerp(a, b, w, one_minus_w):
    return a * one_minus_w + b * w


def _f32_hi(word):
    return lax.bitcast_convert_type(word & _HI16, jnp.float32)


def _f32_lo(word):
    return lax.bitcast_convert_type(lax.shift_left(word, 16), jnp.float32)


def _encode_body(px_hbm, py_hbm, pz_hbm, tp_hbm, out_hbm,
                 px_v, py_v, pz_v, t_v, o_v0, o_v1, sem_p, sem0, sem1):
    cid = lax.axis_index("c")
    sid = lax.axis_index("s")
    wid = sid * 2 + cid
    base_w = wid * _PER_W

    pltpu.async_copy(px_hbm.at[pl.ds(base_w, _PER_W)], px_v, sem_p)
    pltpu.async_copy(py_hbm.at[pl.ds(base_w, _PER_W)], py_v, sem_p)
    pltpu.async_copy(pz_hbm.at[pl.ds(base_w, _PER_W)], pz_v, sem_p)
    pltpu.async_copy(tp_hbm, t_v, sem_p).wait()
    pltpu.make_async_copy(px_hbm.at[pl.ds(base_w, _PER_W)], px_v, sem_p).wait()
    pltpu.make_async_copy(py_hbm.at[pl.ds(base_w, _PER_W)], py_v, sem_p).wait()
    pltpu.make_async_copy(pz_hbm.at[pl.ds(base_w, _PER_W)], pz_v, sem_p).wait()

    def compute_chunk(ch, o_v):
        coff = ch * _CH

        @pl.loop(0, _CH // 16)
        def _(pb):
            off = pb * 16
            x = px_v[pl.ds(coff + off, 16)]
            y = py_v[pl.ds(coff + off, 16)]
            z = pz_v[pl.ds(coff + off, 16)]
            for l in range(_NUM_LEVELS):
                rf = np.float32(_RES[l] - 1)
                rm1 = np.int32(_RES[l] - 1)
                sx = (x + 1.0) * 0.5 * rf
                sy = (y + 1.0) * 0.5 * rf
                sz = (z + 1.0) * 0.5 * rf
                ix = sx.astype(jnp.int32)
                iy = sy.astype(jnp.int32)
                iz = sz.astype(jnp.int32)
                wx = sx - ix.astype(jnp.float32)
                wy = sy - iy.astype(jnp.float32)
                wz = sz - iz.astype(jnp.float32)
                ru = np.uint32(_RES[l] - 1)
                x1 = jnp.minimum((ix + 1).astype(jnp.uint32), ru)
                y1 = jnp.minimum((iy + 1).astype(jnp.uint32), ru)
                z1 = jnp.minimum((iz + 1).astype(jnp.uint32), ru)
                hy0 = iy * _P1
                hy1 = (y1.astype(jnp.int32)) * _P1
                hz0 = iz * _P2
                hz1 = (z1.astype(jnp.int32)) * _P2
                if _RES[l] <= _HASH_ROWS:
                    mx0, mx1 = ix, x1.astype(jnp.int32)
                else:
                    mx0 = ix & _MASK
                    mx1 = x1.astype(jnp.int32) & _MASK
                my0 = hy0 & _MASK
                my1 = hy1 & _MASK
                mz0 = hz0 & _MASK
                mz1 = hz1 & _MASK
                e00 = mx0 ^ my0
                e01 = mx0 ^ my1
                e10 = mx1 ^ my0
                e11 = mx1 ^ my1
                h = [
                    e00 ^ mz0,
                    e00 ^ mz1,
                    e01 ^ mz0,
                    e01 ^ mz1,
                    e10 ^ mz0,
                    e10 ^ mz1,
                    e11 ^ mz0,
                    e11 ^ mz1,
                ]
                tl = t_v.at[pl.ds(l * _HASH_ROWS, _HASH_ROWS)]
                g = [plsc.bitcast(plsc.load_gather(tl, [hj]), jnp.bfloat16)
                     for hj in h]
                fmt = plsc.PackFormat.INTERLEAVED
                wxp = plsc.pack(wx, wx, format=fmt)
                wyp = plsc.pack(wy, wy, format=fmt)
                wzp = plsc.pack(wz, wz, format=fmt)
                owx = 1.0 - wxp
                owy = 1.0 - wyp
                owz = 1.0 - wzp
                c00 = _lerp(g[0], g[1], wxp, owx)
                c01 = _lerp(g[2], g[3], wxp, owx)
                c10 = _lerp(g[4], g[5], wxp, owx)
                c11 = _lerp(g[6], g[7], wxp, owx)
                c0 = _lerp(c00, c01, wyp, owy)
                c1 = _lerp(c10, c11, wyp, owy)
                val = _lerp(c0, c1, wzp, owz)
                w = plsc.bitcast(val, jnp.int32)
                o_v[2 * l, pl.ds(off, 16)] = _f32_lo(w)
                o_v[2 * l + 1, pl.ds(off, 16)] = _f32_hi(w)

    def out_slice(ch):
        return out_hbm.at[:, pl.ds(base_w + ch * _CH, _CH)]

    @pl.loop(0, _NPAIR)
    def _(pair):
        ch0 = pair * 2

        @pl.when(pair > 0)
        def _():
            pltpu.make_async_copy(o_v0, out_slice(ch0), sem0).wait()

        compute_chunk(ch0, o_v0)
        pltpu.async_copy(o_v0, out_slice(ch0), sem0)

        @pl.when(pair > 0)
        def _():
            pltpu.make_async_copy(o_v1, out_slice(ch0 + 1), sem1).wait()

        compute_chunk(ch0 + 1, o_v1)
        pltpu.async_copy(o_v1, out_slice(ch0 + 1), sem1)

    last = 2 * (_NPAIR - 1)
    pltpu.make_async_copy(o_v0, out_slice(last), sem0).wait()
    pltpu.make_async_copy(o_v1, out_slice(last + 1), sem1).wait()


@jax.jit
def _sc_encode(px, py, pz, tp):
    mesh = plsc.VectorSubcoreMesh(core_axis_name="c", subcore_axis_name="s")
    cp = pltpu.CompilerParams()
    for fld, val in (("needs_layout_passes", False),
                     ("use_tc_tiling_on_sc", False)):
        if fld in pltpu.CompilerParams.__dataclass_fields__:
            cp = dataclasses.replace(cp, **{fld: val})
    f = functools.partial(
        pl.kernel,
        compiler_params=cp,
        out_type=jax.ShapeDtypeStruct((2 * _NUM_LEVELS, _N), jnp.float32),
        mesh=mesh,
        scratch_types=[
            pltpu.VMEM((_PER_W,), jnp.float32),
            pltpu.VMEM((_PER_W,), jnp.float32),
            pltpu.VMEM((_PER_W,), jnp.float32),
            pltpu.VMEM((_NUM_LEVELS * _HASH_ROWS,), jnp.int32),
            pltpu.VMEM((2 * _NUM_LEVELS, _CH), jnp.float32),
            pltpu.VMEM((2 * _NUM_LEVELS, _CH), jnp.float32),
            pltpu.SemaphoreType.DMA,
            pltpu.SemaphoreType.DMA,
            pltpu.SemaphoreType.DMA,
        ],
    )(_encode_body)
    out_t = f(px, py, pz, tp)
    return out_t.T


def kernel(positions, table_0, table_1, table_2, table_3, table_4, table_5,
           table_6, table_7, table_8, table_9, table_10, table_11, table_12,
           table_13, table_14, table_15):
    tables = [table_0, table_1, table_2, table_3, table_4, table_5, table_6,
              table_7, table_8, table_9, table_10, table_11, table_12,
              table_13, table_14, table_15]
    px = positions[:, 0]
    py = positions[:, 1]
    pz = positions[:, 2]
    packed = []
    for t in tables:
        bits = lax.bitcast_convert_type(
            t[:_HASH_ROWS].astype(jnp.bfloat16), jnp.uint16)
        word = bits[:, 0].astype(jnp.uint32) | (
            bits[:, 1].astype(jnp.uint32) << 16)
        packed.append(lax.bitcast_convert_type(word, jnp.int32))
    tp = jnp.concatenate(packed)
    return _sc_encode(px, py, pz, tp)


# speedup vs baseline: 1.9657x; 1.0014x over previous
import dataclasses
import functools

import numpy as np
import jax
import jax.numpy as jnp
from jax import lax
from jax.experimental import pallas as pl
from jax.experimental.pallas import tpu as pltpu
from jax.experimental.pallas import tpu_sc as plsc

_NUM_LEVELS = 16
_N = 262144
_HASH_ROWS = 4096
_MASK = np.int32(4095)
_P1 = np.int32(np.uint32(2654435761).view(np.int32))
_P2 = np.int32(805459861)
_RES = [int(np.ceil(16 * 2.0 ** i)) for i in range(_NUM_LEVELS)]

_NW = 32
_PER_W = _N // _NW
_CH = 512
_NPAIR = _PER_W // (2 * _CH)
_HI16 = np.int32(np.uint32(0xFFFF0000).view(np.int32))


def _lerp(a, b, w, one_minus_w):
    return a * one_minus_w + b * w


def _f32_hi(word):
    return lax.bitcast_convert_type(word & _HI16, jnp.float32)


def _f32_lo(word):
    return lax.bitcast_convert_type(lax.shift_left(word, 16), jnp.float32)


def _encode_body(px_hbm, py_hbm, pz_hbm, tp_hbm, out_hbm,
                 px_v, py_v, pz_v, t_v, o_v0, o_v1, sem_p, sem0, sem1):
    cid = lax.axis_index("c")
    sid = lax.axis_index("s")
    wid = sid * 2 + cid
    base_w = wid * _PER_W

    pltpu.async_copy(px_hbm.at[pl.ds(base_w, _PER_W)], px_v, sem_p)
    pltpu.async_copy(py_hbm.at[pl.ds(base_w, _PER_W)], py_v, sem_p)
    pltpu.async_copy(pz_hbm.at[pl.ds(base_w, _PER_W)], pz_v, sem_p)
    pltpu.async_copy(tp_hbm, t_v, sem_p).wait()
    pltpu.make_async_copy(px_hbm.at[pl.ds(base_w, _PER_W)], px_v, sem_p).wait()
    pltpu.make_async_copy(py_hbm.at[pl.ds(base_w, _PER_W)], py_v, sem_p).wait()
    pltpu.make_async_copy(pz_hbm.at[pl.ds(base_w, _PER_W)], pz_v, sem_p).wait()

    def compute_chunk(ch, o_v):
        coff = ch * _CH

        @plsc.parallel_loop(0, _CH // 16)
        def _(pb):
            off = pb * 16
            x = px_v[pl.ds(coff + off, 16)]
            y = py_v[pl.ds(coff + off, 16)]
            z = pz_v[pl.ds(coff + off, 16)]
            for l in range(_NUM_LEVELS):
                rf = np.float32(_RES[l] - 1)
                rm1 = np.int32(_RES[l] - 1)
                sx = (x + 1.0) * 0.5 * rf
                sy = (y + 1.0) * 0.5 * rf
                sz = (z + 1.0) * 0.5 * rf
                ix = sx.astype(jnp.int32)
                iy = sy.astype(jnp.int32)
                iz = sz.astype(jnp.int32)
                wx = sx - ix.astype(jnp.float32)
                wy = sy - iy.astype(jnp.float32)
                wz = sz - iz.astype(jnp.float32)
                ru = np.uint32(_RES[l] - 1)
                x1 = jnp.minimum((ix + 1).astype(jnp.uint32), ru)
                y1 = jnp.minimum((iy + 1).astype(jnp.uint32), ru)
                z1 = jnp.minimum((iz + 1).astype(jnp.uint32), ru)
                hy0 = iy * _P1
                hy1 = (y1.astype(jnp.int32)) * _P1
                hz0 = iz * _P2
                hz1 = (z1.astype(jnp.int32)) * _P2
                if _RES[l] <= _HASH_ROWS:
                    mx0, mx1 = ix, x1.astype(jnp.int32)
                else:
                    mx0 = ix & _MASK
                    mx1 = x1.astype(jnp.int32) & _MASK
                my0 = hy0 & _MASK
                my1 = hy1 & _MASK
                mz0 = hz0 & _MASK
                mz1 = hz1 & _MASK
                e00 = mx0 ^ my0
                e01 = mx0 ^ my1
                e10 = mx1 ^ my0
                e11 = mx1 ^ my1
                h = [
                    e00 ^ mz0,
                    e00 ^ mz1,
                    e01 ^ mz0,
                    e01 ^ mz1,
                    e10 ^ mz0,
                    e10 ^ mz1,
                    e11 ^ mz0,
                    e11 ^ mz1,
                ]
                tl = t_v.at[pl.ds(l * _HASH_ROWS, _HASH_ROWS)]
                g = [plsc.bitcast(plsc.load_gather(tl, [hj]), jnp.bfloat16)
                     for hj in h]
                fmt = plsc.PackFormat.INTERLEAVED
                wxp = plsc.pack(wx, wx, format=fmt)
                wyp = plsc.pack(wy, wy, format=fmt)
                wzp = plsc.pack(wz, wz, format=fmt)
                owx = 1.0 - wxp
                owy = 1.0 - wyp
                owz = 1.0 - wzp
                c00 = _lerp(g[0], g[1], wxp, owx)
                c01 = _lerp(g[2], g[3], wxp, owx)
                c10 = _lerp(g[4], g[5], wxp, owx)
                c11 = _lerp(g[6], g[7], wxp, owx)
                c0 = _lerp(c00, c01, wyp, owy)
                c1 = _lerp(c10, c11, wyp, owy)
                val = _lerp(c0, c1, wzp, owz)
                w = plsc.bitcast(val, jnp.int32)
                o_v[2 * l, pl.ds(off, 16)] = _f32_lo(w)
                o_v[2 * l + 1, pl.ds(off, 16)] = _f32_hi(w)

    def out_slice(ch):
        return out_hbm.at[:, pl.ds(base_w + ch * _CH, _CH)]

    @pl.loop(0, _NPAIR)
    def _(pair):
        ch0 = pair * 2

        @pl.when(pair > 0)
        def _():
            pltpu.make_async_copy(o_v0, out_slice(ch0), sem0).wait()

        compute_chunk(ch0, o_v0)
        pltpu.async_copy(o_v0, out_slice(ch0), sem0)

        @pl.when(pair > 0)
        def _():
            pltpu.make_async_copy(o_v1, out_slice(ch0 + 1), sem1).wait()

        compute_chunk(ch0 + 1, o_v1)
        pltpu.async_copy(o_v1, out_slice(ch0 + 1), sem1)

    last = 2 * (_NPAIR - 1)
    pltpu.make_async_copy(o_v0, out_slice(last), sem0).wait()
    pltpu.make_async_copy(o_v1, out_slice(last + 1), sem1).wait()


@jax.jit
def _sc_encode(px, py, pz, tp):
    mesh = plsc.VectorSubcoreMesh(core_axis_name="c", subcore_axis_name="s")
    cp = pltpu.CompilerParams()
    for fld, val in (("needs_layout_passes", False),
                     ("use_tc_tiling_on_sc", False)):
        if fld in pltpu.CompilerParams.__dataclass_fields__:
            cp = dataclasses.replace(cp, **{fld: val})
    f = functools.partial(
        pl.kernel,
        compiler_params=cp,
        out_type=jax.ShapeDtypeStruct((2 * _NUM_LEVELS, _N), jnp.float32),
        mesh=mesh,
        scratch_types=[
            pltpu.VMEM((_PER_W,), jnp.float32),
            pltpu.VMEM((_PER_W,), jnp.float32),
            pltpu.VMEM((_PER_W,), jnp.float32),
            pltpu.VMEM((_NUM_LEVELS * _HASH_ROWS,), jnp.int32),
            pltpu.VMEM((2 * _NUM_LEVELS, _CH), jnp.float32),
            pltpu.VMEM((2 * _NUM_LEVELS, _CH), jnp.float32),
            pltpu.SemaphoreType.DMA,
            pltpu.SemaphoreType.DMA,
            pltpu.SemaphoreType.DMA,
        ],
    )(_encode_body)
    out_t = f(px, py, pz, tp)
    return out_t.T


def kernel(positions, table_0, table_1, table_2, table_3, table_4, table_5,
           table_6, table_7, table_8, table_9, table_10, table_11, table_12,
           table_13, table_14, table_15):
    tables = [table_0, table_1, table_2, table_3, table_4, table_5, table_6,
              table_7, table_8, table_9, table_10, table_11, table_12,
              table_13, table_14, table_15]
    px = positions[:, 0]
    py = positions[:, 1]
    pz = positions[:, 2]
    packed = []
    for t in tables:
        bits = lax.bitcast_convert_type(
            t[:_HASH_ROWS].astype(jnp.bfloat16), jnp.uint16)
        word = bits[:, 0].astype(jnp.uint32) | (
            bits[:, 1].astype(jnp.uint32) << 16)
        packed.append(lax.bitcast_convert_type(word, jnp.int32))
    tp = jnp.concatenate(packed)
    return _sc_encode(px, py, pz, tp)
